# Initial kernel scaffold; baseline (speedup 1.0000x reference)
import jax
import jax.numpy as jnp
from jax.experimental import pallas as pl


def kernel(x_con, x_cat, E0, E1, E2, gamma1, beta1, W1, b1, W2, b2, Wo, bo):
    raise NotImplementedError("write your pallas kernel here")


# speedup vs baseline: 4.5881x; 4.5881x over previous
import jax
import jax.numpy as jnp
import numpy as np
from jax.experimental import pallas as pl
from jax.experimental.pallas import tpu as pltpu

B = 16384
NUM_CONT = 36
HID = 64
EPS = 1e-5
NCAT = 28
NROWS = 50
NROWS_PAD = 64


def _fused_body(xcat_ref, xcon_ref, tcomb_ref, gamma_ref, beta_ref,
                w1cat_ref, w1con_ref, b1_ref, w2t_ref, b2_ref,
                wot_ref, bo_ref, out_ref):
    col = jax.lax.broadcasted_iota(jnp.int32, (B, NROWS_PAD), 1)
    i0 = xcat_ref[:, 0:1]
    i1 = xcat_ref[:, 1:2]
    i2 = xcat_ref[:, 2:3]
    sel = jnp.where(col < 2, i0, jnp.where(col < 26, i1 + 2, i2 + 26))
    onehot = (sel == col).astype(jnp.float32)
    ecat = jnp.dot(onehot, tcomb_ref[...], preferred_element_type=jnp.float32)
    mean = jnp.mean(ecat, axis=0, keepdims=True)
    meansq = jnp.mean(ecat * ecat, axis=0, keepdims=True)
    var = meansq - mean * mean
    scale = gamma_ref[...] * jax.lax.rsqrt(var + EPS)
    shift = beta_ref[...] - mean * scale
    ecat_n = ecat * scale + shift
    h1 = jnp.dot(ecat_n, w1cat_ref[...], preferred_element_type=jnp.float32)
    h1 = h1 + jnp.dot(xcon_ref[...], w1con_ref[...],
                      preferred_element_type=jnp.float32)
    h1 = jnp.maximum(h1 + b1_ref[...], 0.0)
    h2 = jnp.maximum(
        jnp.dot(h1, w2t_ref[...], preferred_element_type=jnp.float32)
        + b2_ref[...], 0.0)
    out_ref[...] = (jnp.dot(h2, wot_ref[...], preferred_element_type=jnp.float32)
                    + bo_ref[...])


def kernel(x_con, x_cat, E0, E1, E2, gamma1, beta1, W1, b1, W2, b2, Wo, bo):
    x_cat = x_cat.astype(jnp.int32)
    tcomb = jnp.zeros((NROWS_PAD, NCAT), dtype=jnp.float32)
    tcomb = tcomb.at[0:2, 0:4].set(E0)
    tcomb = tcomb.at[2:26, 4:16].set(E1)
    tcomb = tcomb.at[26:50, 16:28].set(E2)
    w1cat = W1[:, :NCAT].T
    w1con = W1[:, NCAT:].T
    out = pl.pallas_call(
        _fused_body,
        out_shape=jax.ShapeDtypeStruct((B, 1), jnp.float32),
    )(x_cat, x_con, tcomb,
      gamma1.reshape(1, NCAT), beta1.reshape(1, NCAT),
      w1cat, w1con, b1.reshape(1, HID),
      W2.T, b2.reshape(1, HID),
      Wo.T, bo.reshape(1, 1))
    return out
